# baseline (speedup 1.0000x reference)
import functools

import jax
import jax.numpy as jnp
from jax import lax
from jax.experimental import pallas as pl
from jax.experimental.pallas import tpu as pltpu
from jax.experimental.pallas import tpu_sc as plsc

_CHUNK = 104
_CHUNKS_PER_ROW = 2
_SPAD = _CHUNK * _CHUNKS_PER_ROW
_NBUF = 4
_UNROLL = 8
_LANES = 16


@functools.cache
def _make_sc_kernel(B, E):
    info = plsc.get_sparse_core_info()
    NC, NS = info.num_cores, info.num_subcores
    NW = NC * NS
    rows_w = B // NW
    nch = rows_w * _CHUNKS_PER_ROW
    nvec = E // _LANES
    mesh = plsc.VectorSubcoreMesh(core_axis_name="c", subcore_axis_name="s")

    @functools.partial(
        pl.kernel,
        out_type=jax.ShapeDtypeStruct((B, E), jnp.float32),
        mesh=mesh,
        compiler_params=pltpu.CompilerParams(use_tc_tiling_on_sc=False),
        scratch_types=[
            pltpu.VMEM((rows_w, _CHUNKS_PER_ROW, _CHUNK), jnp.int32),
            pltpu.VMEM((_NBUF, _CHUNK, E), jnp.float32),
            pltpu.VMEM((rows_w, E), jnp.float32),
            pltpu.SemaphoreType.DMA,
            pltpu.SemaphoreType.DMA,
            pltpu.SemaphoreType.DMA,
            pltpu.SemaphoreType.DMA,
        ],
    )
    def bow(idx_hbm, table_hbm, out_hbm, idx_v, buf_v, out_v, s0, s1, s2, s3):
        sems = (s0, s1, s2, s3)
        wid = lax.axis_index("s") * NC + lax.axis_index("c")
        base = wid * rows_w
        pltpu.sync_copy(idx_hbm.at[pl.ds(base, rows_w)], idx_v)

        def gather(row, half, slot):
            return pltpu.make_async_copy(
                table_hbm.at[idx_v.at[row, half]], buf_v.at[slot], sems[slot])

        def reduce_into(slot, acc):
            def body(jj, a):
                a = list(a)
                for u in range(_UNROLL):
                    j = jj * _UNROLL + u
                    for k in range(nvec):
                        a[k] = jnp.maximum(
                            a[k], buf_v[slot, j, pl.ds(k * _LANES, _LANES)])
                return tuple(a)
            return lax.fori_loop(0, _CHUNK // _UNROLL, body, acc)

        neg_inf = jnp.full((_LANES,), -jnp.inf, dtype=jnp.float32)

        def finalize(row, acc):
            for k in range(nvec):
                x = acc[k]
                out_v[row, pl.ds(k * _LANES, _LANES)] = (
                    1.0 - 2.0 / (jnp.exp(2.0 * x) + 1.0))

        for i in range(_NBUF):
            gather(i // _CHUNKS_PER_ROW, i % _CHUNKS_PER_ROW, i).start()

        def step(c0, last):
            rbase = c0 // _CHUNKS_PER_ROW
            acc = (neg_inf,) * nvec
            for i in range(_NBUF):
                row = rbase + i // _CHUNKS_PER_ROW
                half = i % _CHUNKS_PER_ROW
                gather(row, half, i).wait()
                acc = reduce_into(i, acc)
                if half == _CHUNKS_PER_ROW - 1:
                    finalize(row, acc)
                    acc = (neg_inf,) * nvec
                if not last:
                    gather(row + _NBUF // _CHUNKS_PER_ROW, half, i).start()

        @pl.loop(0, nch - _NBUF, step=_NBUF)
        def _(c0):
            step(c0, False)

        step(nch - _NBUF, True)
        pltpu.sync_copy(out_v, out_hbm.at[pl.ds(base, rows_w)])

    return bow


def kernel(input, emb_table):
    B, S = input.shape
    _, E = emb_table.shape
    idx = input.astype(jnp.int32)
    idx = jnp.concatenate(
        [idx, jnp.broadcast_to(idx[:, :1], (B, _SPAD - S))], axis=1)
    idx = idx.reshape(B, _CHUNKS_PER_ROW, _CHUNK)
    return _make_sc_kernel(B, E)(idx, emb_table)
